# baseline (device time: 18762 ns/iter reference)
import jax
import jax.numpy as jnp
from jax import lax
from jax.experimental import pallas as pl
from jax.experimental.pallas import tpu as pltpu

N_DEV = 4

QSCALE = 5.0


def kernel(x, w_mat):
    m_glob, k_shard = x.shape
    n = w_mat.shape[1]
    m_blk = m_glob // N_DEV
    n_half = n // 2

    def body(x_hbm_ref, w_hbm_ref, out_hbm_ref, xv_ref, sendb_ref, xt_ref,
             wv_ref, acc_ref, send_sems, recv_sems, wdma_sems, xdma_sems,
             odma_sems):
        my = lax.axis_index("i")

        xdmas = []
        for d in range(N_DEV):
            blk_dev = lax.rem(my + d, N_DEV)
            xdmas.append(
                pltpu.make_async_copy(
                    x_hbm_ref.at[pl.ds(blk_dev * m_blk, m_blk), :],
                    xv_ref.at[d],
                    xdma_sems.at[d],
                )
            )
        for d in range(1, N_DEV):
            xdmas[d].start()
        xdmas[0].start()

        wdmas = []
        for d in range(N_DEV):
            src_dev = lax.rem(my - d + N_DEV, N_DEV)
            wdmas.append(
                pltpu.make_async_copy(
                    w_hbm_ref.at[pl.ds(src_dev * k_shard, k_shard), :],
                    wv_ref.at[d],
                    wdma_sems.at[d],
                )
            )
        wdmas[0].start()

        for d in range(1, N_DEV):
            xdmas[d].wait()
            sendb_ref[d - 1, :, :] = jnp.clip(
                jnp.round(xv_ref[d] * (127.0 / QSCALE)), -127.0, 127.0
            ).astype(jnp.int8)

        barrier_sem = pltpu.get_barrier_semaphore()
        for d in range(1, N_DEV):
            peer = lax.rem(my + d, N_DEV)
            pl.semaphore_signal(
                barrier_sem, inc=1,
                device_id=(peer,), device_id_type=pl.DeviceIdType.MESH,
            )
        pl.semaphore_wait(barrier_sem, N_DEV - 1)

        rdmas = []
        for d in range(1, N_DEV):
            dst_dev = lax.rem(my + d, N_DEV)
            rdma = pltpu.make_async_remote_copy(
                src_ref=sendb_ref.at[d - 1],
                dst_ref=xt_ref.at[d - 1],
                send_sem=send_sems.at[d - 1],
                recv_sem=recv_sems.at[d - 1],
                device_id=(dst_dev,),
                device_id_type=pl.DeviceIdType.MESH,
            )
            rdma.start()
            rdmas.append(rdma)

        for d in range(1, N_DEV):
            wdmas[d].start()

        wdmas[0].wait()
        xdmas[0].wait()
        acc_ref[...] = jnp.dot(
            xv_ref[0].astype(jnp.bfloat16),
            wv_ref[0].astype(jnp.bfloat16),
            preferred_element_type=jnp.float32,
        )

        for d in range(1, N_DEV - 1):
            wdmas[d].wait()
            wb = (wv_ref[d] * (QSCALE / 127.0)).astype(jnp.bfloat16)
            rdmas[d - 1].wait_recv()
            acc_ref[...] += jnp.dot(
                xt_ref[d - 1].astype(jnp.bfloat16), wb,
                preferred_element_type=jnp.float32,
            )

        d = N_DEV - 1
        wdmas[d].wait()
        wb = (wv_ref[d] * (QSCALE / 127.0)).astype(jnp.bfloat16)
        rdmas[d - 1].wait_recv()
        xb = xt_ref[d - 1].astype(jnp.bfloat16)

        acc_ref[:, 0:n_half] += jnp.dot(
            xb, wb[:, 0:n_half], preferred_element_type=jnp.float32
        )
        odma0 = pltpu.make_async_copy(
            acc_ref.at[:, pl.ds(0, n_half)],
            out_hbm_ref.at[:, pl.ds(0, n_half)],
            odma_sems.at[0],
        )
        odma0.start()

        acc_ref[:, n_half:n] += jnp.dot(
            xb, wb[:, n_half:n], preferred_element_type=jnp.float32
        )
        odma1 = pltpu.make_async_copy(
            acc_ref.at[:, pl.ds(n_half, n_half)],
            out_hbm_ref.at[:, pl.ds(n_half, n_half)],
            odma_sems.at[1],
        )
        odma1.start()

        odma0.wait()
        odma1.wait()
        for d in range(1, N_DEV):
            rdmas[d - 1].wait_send()

    return pl.pallas_call(
        body,
        out_shape=jax.ShapeDtypeStruct((m_blk, n), jnp.float32),
        in_specs=[
            pl.BlockSpec(memory_space=pltpu.MemorySpace.HBM),
            pl.BlockSpec(memory_space=pltpu.MemorySpace.HBM),
        ],
        out_specs=pl.BlockSpec(memory_space=pltpu.MemorySpace.HBM),
        scratch_shapes=[
            pltpu.VMEM((N_DEV, m_blk, k_shard), jnp.float32),
            pltpu.VMEM((N_DEV - 1, m_blk, k_shard), jnp.int8),
            pltpu.VMEM((N_DEV - 1, m_blk, k_shard), jnp.int8),
            pltpu.VMEM((N_DEV, k_shard, n), jnp.float32),
            pltpu.VMEM((m_blk, n), jnp.float32),
            pltpu.SemaphoreType.DMA((N_DEV - 1,)),
            pltpu.SemaphoreType.DMA((N_DEV - 1,)),
            pltpu.SemaphoreType.DMA((N_DEV,)),
            pltpu.SemaphoreType.DMA((N_DEV,)),
            pltpu.SemaphoreType.DMA((2,)),
        ],
        compiler_params=pltpu.CompilerParams(collective_id=0),
    )(x, w_mat)


# device time: 17126 ns/iter; 1.0955x vs baseline; 1.0955x over previous
import jax
import jax.numpy as jnp
from jax import lax
from jax.experimental import pallas as pl
from jax.experimental.pallas import tpu as pltpu

N_DEV = 4

QSCALE = 5.0


def kernel(x, w_mat):
    m_glob, k_shard = x.shape
    n = w_mat.shape[1]
    m_blk = m_glob // N_DEV

    def body(x_ref, w_hbm_ref, out_ref, sendb_ref, xt_ref, wv_ref,
             send_sems, recv_sems, wdma_sems):
        my = lax.axis_index("i")

        wdmas = []
        for d in range(N_DEV):
            src_dev = lax.rem(my - d + N_DEV, N_DEV)
            wdmas.append(
                pltpu.make_async_copy(
                    w_hbm_ref.at[pl.ds(src_dev * k_shard, k_shard), :],
                    wv_ref.at[d],
                    wdma_sems.at[d],
                )
            )
        wdmas[0].start()

        for d in range(1, N_DEV):
            dst_dev = lax.rem(my + d, N_DEV)
            blk = x_ref[pl.ds(dst_dev * m_blk, m_blk), :]
            sendb_ref[d - 1, :, :] = jnp.clip(
                jnp.round(blk * (127.0 / QSCALE)), -127.0, 127.0
            ).astype(jnp.int8)

        barrier_sem = pltpu.get_barrier_semaphore()
        for d in range(1, N_DEV):
            peer = lax.rem(my + d, N_DEV)
            pl.semaphore_signal(
                barrier_sem, inc=1,
                device_id=(peer,), device_id_type=pl.DeviceIdType.MESH,
            )
        pl.semaphore_wait(barrier_sem, N_DEV - 1)

        rdmas = []
        for d in range(1, N_DEV):
            dst_dev = lax.rem(my + d, N_DEV)
            rdma = pltpu.make_async_remote_copy(
                src_ref=sendb_ref.at[d - 1],
                dst_ref=xt_ref.at[d - 1],
                send_sem=send_sems.at[d - 1],
                recv_sem=recv_sems.at[d - 1],
                device_id=(dst_dev,),
                device_id_type=pl.DeviceIdType.MESH,
            )
            rdma.start()
            rdmas.append(rdma)

        for d in range(1, N_DEV):
            wdmas[d].start()

        wdmas[0].wait()
        out_ref[...] = jnp.dot(
            x_ref[pl.ds(my * m_blk, m_blk), :].astype(jnp.bfloat16),
            wv_ref[0].astype(jnp.bfloat16),
            preferred_element_type=jnp.float32,
        )

        for d in range(1, N_DEV):
            wdmas[d].wait()
            wb = (wv_ref[d] * (QSCALE / 127.0)).astype(jnp.bfloat16)
            rdmas[d - 1].wait_recv()
            out_ref[...] += jnp.dot(
                xt_ref[d - 1].astype(jnp.bfloat16), wb,
                preferred_element_type=jnp.float32,
            )

        for d in range(1, N_DEV):
            rdmas[d - 1].wait_send()

    return pl.pallas_call(
        body,
        out_shape=jax.ShapeDtypeStruct((m_blk, n), jnp.float32),
        in_specs=[
            pl.BlockSpec(memory_space=pltpu.VMEM),
            pl.BlockSpec(memory_space=pltpu.MemorySpace.HBM),
        ],
        out_specs=pl.BlockSpec(memory_space=pltpu.VMEM),
        scratch_shapes=[
            pltpu.VMEM((N_DEV - 1, m_blk, k_shard), jnp.int8),
            pltpu.VMEM((N_DEV - 1, m_blk, k_shard), jnp.int8),
            pltpu.VMEM((N_DEV, k_shard, n), jnp.float32),
            pltpu.SemaphoreType.DMA((N_DEV - 1,)),
            pltpu.SemaphoreType.DMA((N_DEV - 1,)),
            pltpu.SemaphoreType.DMA((N_DEV,)),
        ],
        compiler_params=pltpu.CompilerParams(collective_id=0),
    )(x, w_mat)
